# baseline (speedup 1.0000x reference)
import jax
import jax.numpy as jnp
from jax.experimental import pallas as pl
from jax.experimental.pallas import tpu as pltpu

B_, I_, F_, K_ = 2048, 8, 2048, 16
RB = 512


def _fused_kernel(x_ref, a_ref, b_ref, out_ref, inner_ref, w_ref, m_ref,
                  s_ref):
    @pl.when(pl.program_id(1) == 0)
    def _():
        cols = []
        for i in range(I_):
            a = a_ref[i]
            cols.append(a * jax.lax.rsqrt(jnp.sum(a * a, axis=0,
                                                  keepdims=True)))
        w_ref[...] = jnp.concatenate(cols, axis=1)
        row_inst = jax.lax.broadcasted_iota(jnp.int32, (RB, I_ * K_), 0) % I_
        lane_inst = jax.lax.broadcasted_iota(jnp.int32, (RB, I_ * K_), 1) // K_
        m_ref[...] = (row_inst == lane_inst).astype(jnp.float32)
        sel_row = jax.lax.broadcasted_iota(jnp.int32, (I_ * K_, K_), 0) % K_
        sel_col = jax.lax.broadcasted_iota(jnp.int32, (I_ * K_, K_), 1)
        s_ref[...] = (sel_row == sel_col).astype(jnp.float32)

    inner_full = jnp.dot(x_ref[...], w_ref[...],
                         preferred_element_type=jnp.float32)
    inner_masked = inner_full * m_ref[...]
    out_ref[...] = jnp.dot(inner_masked, b_ref[...],
                           preferred_element_type=jnp.float32)
    inner_ref[...] = jnp.dot(inner_masked, s_ref[...],
                             preferred_element_type=jnp.float32)


def kernel(x, A, B):
    xf = x.reshape(B_ * I_, F_)
    bf = B.reshape(I_ * K_, F_)
    nr = (B_ * I_) // RB
    out2, inner2 = pl.pallas_call(
        _fused_kernel,
        grid=(1, nr),
        in_specs=[
            pl.BlockSpec((RB, F_), lambda c, b: (b, 0)),
            pl.BlockSpec((I_, F_, K_), lambda c, b: (0, 0, 0)),
            pl.BlockSpec((I_ * K_, F_), lambda c, b: (0, 0)),
        ],
        out_specs=[
            pl.BlockSpec((RB, F_), lambda c, b: (b, 0)),
            pl.BlockSpec((RB, K_), lambda c, b: (b, 0)),
        ],
        out_shape=[
            jax.ShapeDtypeStruct((B_ * I_, F_), jnp.float32),
            jax.ShapeDtypeStruct((B_ * I_, K_), jnp.float32),
        ],
        scratch_shapes=[
            pltpu.VMEM((F_, I_ * K_), jnp.float32),
            pltpu.VMEM((RB, I_ * K_), jnp.float32),
            pltpu.VMEM((I_ * K_, K_), jnp.float32),
        ],
    )(xf, A, bf)
    out = out2.reshape(B_, I_, F_)
    inner = inner2.reshape(B_, I_, K_)
    return (out, inner)


# speedup vs baseline: 1.0764x; 1.0764x over previous
import jax
import jax.numpy as jnp
from jax.experimental import pallas as pl
from jax.experimental.pallas import tpu as pltpu

B_, I_, F_, K_ = 2048, 8, 2048, 16
RB = 1024


def _fused_kernel(x0_ref, x1_ref, a_ref, b_ref, out_ref, inner_ref, w_ref,
                  m_ref, s_ref):
    @pl.when(pl.program_id(1) == 0)
    def _():
        cols = []
        for i in range(I_):
            a = a_ref[i]
            cols.append(a * jax.lax.rsqrt(jnp.sum(a * a, axis=0,
                                                  keepdims=True)))
        w_ref[...] = jnp.concatenate(cols, axis=1)
        row_inst = jax.lax.broadcasted_iota(jnp.int32, (RB, I_ * K_), 0) % I_
        lane_inst = jax.lax.broadcasted_iota(jnp.int32, (RB, I_ * K_), 1) // K_
        m_ref[...] = (row_inst == lane_inst).astype(jnp.float32)
        sel_row = jax.lax.broadcasted_iota(jnp.int32, (I_ * K_, K_), 0) % K_
        sel_col = jax.lax.broadcasted_iota(jnp.int32, (I_ * K_, K_), 1)
        s_ref[...] = (sel_row == sel_col).astype(jnp.float32)

    h = F_ // 2
    inner_full = (jnp.dot(x0_ref[...], w_ref[:h, :],
                          preferred_element_type=jnp.float32) +
                  jnp.dot(x1_ref[...], w_ref[h:, :],
                          preferred_element_type=jnp.float32))
    inner_masked = inner_full * m_ref[...]
    out_ref[...] = jnp.dot(inner_masked, b_ref[...],
                           preferred_element_type=jnp.float32)
    inner_ref[...] = jnp.dot(inner_masked, s_ref[...],
                             preferred_element_type=jnp.float32)


def kernel(x, A, B):
    xf = x.reshape(B_ * I_, F_)
    bf = B.reshape(I_ * K_, F_)
    nr = (B_ * I_) // RB
    out2, inner2 = pl.pallas_call(
        _fused_kernel,
        grid=(1, nr),
        in_specs=[
            pl.BlockSpec((RB, F_ // 2), lambda c, b: (b, 0)),
            pl.BlockSpec((RB, F_ // 2), lambda c, b: (b, 1)),
            pl.BlockSpec((I_, F_, K_), lambda c, b: (0, 0, 0)),
            pl.BlockSpec((I_ * K_, F_), lambda c, b: (0, 0)),
        ],
        out_specs=[
            pl.BlockSpec((RB, F_), lambda c, b: (b, 0)),
            pl.BlockSpec((RB, K_), lambda c, b: (b, 0)),
        ],
        compiler_params=pltpu.CompilerParams(
            vmem_limit_bytes=100 * 1024 * 1024),
        out_shape=[
            jax.ShapeDtypeStruct((B_ * I_, F_), jnp.float32),
            jax.ShapeDtypeStruct((B_ * I_, K_), jnp.float32),
        ],
        scratch_shapes=[
            pltpu.VMEM((F_, I_ * K_), jnp.float32),
            pltpu.VMEM((RB, I_ * K_), jnp.float32),
            pltpu.VMEM((I_ * K_, K_), jnp.float32),
        ],
    )(xf, xf, A, bf)
    out = out2.reshape(B_, I_, F_)
    inner = inner2.reshape(B_, I_, K_)
    return (out, inner)


# speedup vs baseline: 1.0769x; 1.0004x over previous
import jax
import jax.numpy as jnp
from jax.experimental import pallas as pl
from jax.experimental.pallas import tpu as pltpu

B_, I_, F_, K_ = 2048, 8, 2048, 16
RB = 1024


def _fused_kernel(x_ref, a_ref, b_ref, out_ref, inner_ref, w_ref, m_ref,
                  s_ref):
    @pl.when(pl.program_id(1) == 0)
    def _():
        cols = []
        for i in range(I_):
            a = a_ref[i]
            cols.append(a * jax.lax.rsqrt(jnp.sum(a * a, axis=0,
                                                  keepdims=True)))
        w_ref[...] = jnp.concatenate(cols, axis=1)
        row_inst = jax.lax.broadcasted_iota(jnp.int32, (RB, I_ * K_), 0) % I_
        lane_inst = jax.lax.broadcasted_iota(jnp.int32, (RB, I_ * K_), 1) // K_
        m_ref[...] = (row_inst == lane_inst).astype(jnp.float32)
        sel_row = jax.lax.broadcasted_iota(jnp.int32, (I_ * K_, K_), 0) % K_
        sel_col = jax.lax.broadcasted_iota(jnp.int32, (I_ * K_, K_), 1)
        s_ref[...] = (sel_row == sel_col).astype(jnp.float32)

    inner_full = jnp.dot(x_ref[...], w_ref[...],
                         preferred_element_type=jnp.float32)
    inner_masked = inner_full * m_ref[...]
    out_ref[...] = jnp.dot(inner_masked, b_ref[...],
                           preferred_element_type=jnp.float32)
    inner_ref[...] = jnp.dot(inner_masked, s_ref[...],
                             preferred_element_type=jnp.float32)


def kernel(x, A, B):
    xf = x.reshape(B_ * I_, F_)
    bf = B.reshape(I_ * K_, F_)
    nr = (B_ * I_) // RB
    out2, inner2 = pl.pallas_call(
        _fused_kernel,
        grid=(1, nr),
        in_specs=[
            pl.BlockSpec((RB, F_), lambda c, b: (b, 0)),
            pl.BlockSpec((I_, F_, K_), lambda c, b: (0, 0, 0)),
            pl.BlockSpec((I_ * K_, F_), lambda c, b: (0, 0)),
        ],
        out_specs=[
            pl.BlockSpec((RB, F_), lambda c, b: (b, 0)),
            pl.BlockSpec((RB, K_), lambda c, b: (b, 0)),
        ],
        out_shape=[
            jax.ShapeDtypeStruct((B_ * I_, F_), jnp.float32),
            jax.ShapeDtypeStruct((B_ * I_, K_), jnp.float32),
        ],
        scratch_shapes=[
            pltpu.VMEM((F_, I_ * K_), jnp.float32),
            pltpu.VMEM((RB, I_ * K_), jnp.float32),
            pltpu.VMEM((I_ * K_, K_), jnp.float32),
        ],
    )(xf, A, bf)
    out = out2.reshape(B_, I_, F_)
    inner = inner2.reshape(B_, I_, K_)
    return (out, inner)
